# Initial kernel scaffold; baseline (speedup 1.0000x reference)
import jax
import jax.numpy as jnp
from jax.experimental import pallas as pl


def kernel(source_nodes, destination_nodes, timestamps, edge_idxs, node_features, edge_features, memory, last_update, t_w1, t_b1, t_w2, t_b2, msg_w1, msg_b1, msg_w2, msg_b2, gru_wih, gru_whh, gru_bih, gru_bhh, attn_in_w, attn_in_b, attn_out_w, attn_out_b, emb_w, emb_b, proj_w, proj_b, lp_w1, lp_b1, lp_w2, lp_b2, lp_w3, lp_b3):
    raise NotImplementedError("write your pallas kernel here")


# speedup vs baseline: 3.9686x; 3.9686x over previous
import functools

import jax
import jax.numpy as jnp
from jax import lax
from jax.experimental import pallas as pl
from jax.experimental.pallas import tpu as pltpu
from jax.experimental.pallas import tpu_sc as plsc

_B = 4096
_N = 100000
_F = 128
_NC = 2
_NS = 16
_NW = _NC * _NS
_EW = _B // _NW
_TRS = 56
_TROWS = _TRS * _NS
_RNG = _TRS * 128

_mesh = plsc.VectorSubcoreMesh(core_axis_name="c", subcore_axis_name="s")
_params = pltpu.CompilerParams(needs_layout_passes=False)


@functools.partial(
    pl.kernel,
    out_type=[
        jax.ShapeDtypeStruct((_B, _F), jnp.float32),
        jax.ShapeDtypeStruct((_B, _F), jnp.float32),
        jax.ShapeDtypeStruct((_B, _F), jnp.float32),
        jax.ShapeDtypeStruct((_B, _F), jnp.float32),
        jax.ShapeDtypeStruct((_B, _F), jnp.float32),
        jax.ShapeDtypeStruct((_B,), jnp.float32),
        jax.ShapeDtypeStruct((_B,), jnp.float32),
        jax.ShapeDtypeStruct((_B,), jnp.int32),
        jax.ShapeDtypeStruct((_B,), jnp.int32),
        jax.ShapeDtypeStruct((_B,), jnp.int32),
    ],
    mesh=_mesh,
    compiler_params=_params,
    scratch_types=[
        pltpu.VMEM((_B,), jnp.int32),
        pltpu.VMEM((_TRS, 128), jnp.int32),
        pltpu.HBM((_TROWS, 128), jnp.int32),
        pltpu.VMEM((_EW,), jnp.int32),
        pltpu.VMEM((_EW,), jnp.int32),
        pltpu.VMEM((_EW,), jnp.int32),
        pltpu.VMEM((_EW,), jnp.int32),
        pltpu.VMEM((_EW, _F), jnp.float32),
        pltpu.VMEM((_EW, _F), jnp.float32),
        pltpu.VMEM((_EW, _F), jnp.float32),
        pltpu.VMEM((_EW, _F), jnp.float32),
        pltpu.VMEM((_EW, _F), jnp.float32),
        pltpu.VMEM((_EW, 128), jnp.float32),
        pltpu.VMEM((_EW, 128), jnp.int32),
        pltpu.VMEM((_EW,), jnp.float32),
        pltpu.VMEM((_EW,), jnp.float32),
        pltpu.VMEM((_EW,), jnp.int32),
        pltpu.VMEM((_EW,), jnp.int32),
        pltpu.VMEM((_EW,), jnp.int32),
        pltpu.SemaphoreType.DMA,
        pltpu.SemaphoreType.DMA,
    ],
)
def _sc_gather_resolve(src_hbm, dst_hbm, eidx_hbm, nf_hbm, ef_hbm, mem_hbm,
                       lu2d_hbm,
                       nfs_o, nfd_o, ef_o, mems_o, memd_o, lus_o, lud_o,
                       js_o, jd_o, jdp_o,
                       src_all, tbl_loc, tbl_sh, src_v, dst_v, eidx_v, row_v,
                       rb0, rb1, rb2, rb3, rb4, lrx, trx,
                       lusb, ludb, jsb, jdb, jdpb, sem, sem2):
    c = lax.axis_index("c")
    s = lax.axis_index("s")
    wid = s * _NC + c
    base = wid * _EW
    lo = s * _RNG
    iota = lax.iota(jnp.int32, 16)

    pltpu.sync_copy(src_hbm, src_all)

    def zero_body(k, _):
        for q in range(8):
            tbl_loc[k, pl.ds(q * 16, 16)] = jnp.zeros((16,), jnp.int32)
        return 0
    lax.fori_loop(0, _TRS, zero_body, 0)

    def scat_body(k, _):
        ids = src_all[pl.ds(k * 16, 16)]
        inr = (ids >= lo) & (ids < lo + _RNG)
        ev1 = k * 16 + iota + 1
        loc = jnp.where(inr, ids - lo, 0)
        r = loc >> 7
        cc = loc & 127
        for l in range(16):
            plsc.store_scatter(tbl_loc, [r, cc], ev1, mask=inr & (iota == l))
        return 0
    lax.fori_loop(0, _B // 16, scat_body, 0)

    pltpu.sync_copy(tbl_loc, tbl_sh.at[pl.ds(s * _TRS, _TRS)])
    plsc.subcore_barrier()

    pltpu.sync_copy(src_hbm.at[pl.ds(base, _EW)], src_v)
    pltpu.sync_copy(dst_hbm.at[pl.ds(base, _EW)], dst_v)
    pltpu.sync_copy(eidx_hbm.at[pl.ds(base, _EW)], eidx_v)

    cps = [
        pltpu.async_copy(nf_hbm.at[src_v], rb0, sem),
        pltpu.async_copy(nf_hbm.at[dst_v], rb1, sem),
        pltpu.async_copy(ef_hbm.at[eidx_v], rb2, sem),
        pltpu.async_copy(mem_hbm.at[src_v], rb3, sem),
        pltpu.async_copy(mem_hbm.at[dst_v], rb4, sem),
    ]

    def lookup_rows(idx_v, table, stage):
        for t in range(_EW // 16):
            row_v[pl.ds(t * 16, 16)] = idx_v[pl.ds(t * 16, 16)] >> 7
        pltpu.async_copy(table.at[row_v], stage, sem2).wait()

    lookup_rows(src_v, lu2d_hbm, lrx)
    for t in range(_EW // 16):
        sl = pl.ds(t * 16, 16)
        lusb[sl] = plsc.load_gather(lrx, [t * 16 + iota, src_v[sl] & 127])
    lookup_rows(dst_v, lu2d_hbm, lrx)
    for t in range(_EW // 16):
        sl = pl.ds(t * 16, 16)
        ludb[sl] = plsc.load_gather(lrx, [t * 16 + iota, dst_v[sl] & 127])
    lookup_rows(src_v, tbl_sh, trx)
    for t in range(_EW // 16):
        sl = pl.ds(t * 16, 16)
        jsp = plsc.load_gather(trx, [t * 16 + iota, src_v[sl] & 127])
        jsb[sl] = jsp - 1
    lookup_rows(dst_v, tbl_sh, trx)
    for t in range(_EW // 16):
        sl = pl.ds(t * 16, 16)
        jdp = plsc.load_gather(trx, [t * 16 + iota, dst_v[sl] & 127])
        jdpb[sl] = jdp
        own = base + t * 16 + iota
        jdb[sl] = jnp.where(jdp > 0, jdp - 1, own)

    for cp in cps:
        cp.wait()

    pltpu.sync_copy(rb0, nfs_o.at[pl.ds(base, _EW)])
    pltpu.sync_copy(rb1, nfd_o.at[pl.ds(base, _EW)])
    pltpu.sync_copy(rb2, ef_o.at[pl.ds(base, _EW)])
    pltpu.sync_copy(rb3, mems_o.at[pl.ds(base, _EW)])
    pltpu.sync_copy(rb4, memd_o.at[pl.ds(base, _EW)])
    pltpu.sync_copy(lusb, lus_o.at[pl.ds(base, _EW)])
    pltpu.sync_copy(ludb, lud_o.at[pl.ds(base, _EW)])
    pltpu.sync_copy(jsb, js_o.at[pl.ds(base, _EW)])
    pltpu.sync_copy(jdb, jd_o.at[pl.ds(base, _EW)])
    pltpu.sync_copy(jdpb, jdp_o.at[pl.ds(base, _EW)])


@functools.partial(
    pl.kernel,
    out_type=[
        jax.ShapeDtypeStruct((_B, _F), jnp.float32),
        jax.ShapeDtypeStruct((_B, _F), jnp.float32),
        jax.ShapeDtypeStruct((_B,), jnp.float32),
        jax.ShapeDtypeStruct((_B,), jnp.float32),
    ],
    mesh=_mesh,
    compiler_params=_params,
    scratch_types=[
        pltpu.VMEM((_EW,), jnp.int32),
        pltpu.VMEM((_EW,), jnp.int32),
        pltpu.VMEM((_B,), jnp.float32),
        pltpu.VMEM((_EW, _F), jnp.float32),
        pltpu.VMEM((_EW, _F), jnp.float32),
        pltpu.VMEM((_EW,), jnp.float32),
        pltpu.VMEM((_EW,), jnp.float32),
        pltpu.SemaphoreType.DMA,
    ],
)
def _sc_permute_rows(tab_hbm, ts_hbm, js_hbm, jd_hbm,
                     nhs_o, nhd_o, tjs_o, tjd_o,
                     js_v, jd_v, ts_v, rb0, rb1, tsb0, tsb1, sem):
    c = lax.axis_index("c")
    s = lax.axis_index("s")
    base = (s * _NC + c) * _EW
    pltpu.sync_copy(js_hbm.at[pl.ds(base, _EW)], js_v)
    pltpu.sync_copy(jd_hbm.at[pl.ds(base, _EW)], jd_v)
    pltpu.sync_copy(ts_hbm, ts_v)
    cps = [
        pltpu.async_copy(tab_hbm.at[js_v], rb0, sem),
        pltpu.async_copy(tab_hbm.at[jd_v], rb1, sem),
    ]
    for t in range(_EW // 16):
        sl = pl.ds(t * 16, 16)
        tsb0[sl] = plsc.load_gather(ts_v, [js_v[sl]])
        tsb1[sl] = plsc.load_gather(ts_v, [jd_v[sl]])
    for cp in cps:
        cp.wait()
    pltpu.sync_copy(rb0, nhs_o.at[pl.ds(base, _EW)])
    pltpu.sync_copy(rb1, nhd_o.at[pl.ds(base, _EW)])
    pltpu.sync_copy(tsb0, tjs_o.at[pl.ds(base, _EW)])
    pltpu.sync_copy(tsb1, tjd_o.at[pl.ds(base, _EW)])


def _tc_msg_gru(mem_s_r, nfd_r, ef_r, ts_r, lu_r,
                w1a_r, w1b_r, w1c_r, wts_r, wdt_r, b1_r, w2_r, b2_r,
                wih_r, whh_r, bih_r, bhh_r, out_r):
    mem_s = mem_s_r[...]
    ts = ts_r[...]
    h = (jnp.dot(mem_s, w1a_r[...]) + jnp.dot(nfd_r[...], w1b_r[...])
         + jnp.dot(ef_r[...], w1c_r[...])
         + ts * wts_r[...] + (ts - lu_r[...]) * wdt_r[...] + b1_r[...])
    m = jnp.dot(jax.nn.relu(h), w2_r[...]) + b2_r[...]
    gi = jnp.dot(m, wih_r[...]) + bih_r[...]
    gh = jnp.dot(mem_s, whh_r[...]) + bhh_r[...]
    r = jax.nn.sigmoid(gi[:, :128] + gh[:, :128])
    z = jax.nn.sigmoid(gi[:, 128:256] + gh[:, 128:256])
    n = jnp.tanh(gi[:, 256:384] + r * gh[:, 256:384])
    out_r[...] = (1.0 - z) * n + z * mem_s


def _tc_embed_lp(nhs_r, nhd_r, tjs_r, tjd_r, jdp_r, memd_r, lud_r, nfs_r,
                 nfd_r, ts_r,
                 wvT_r, bv_r, owT_r, ob_r, ewT_r, eb_r, pwT_r, pb_r,
                 tw1_r, tb1_r, tw2T_r, tb2_r, aT_r, bT_r, lb1_r,
                 w2T_r, lb2_r, w3T_r, lb3_r, out_r):
    C1 = jnp.dot(wvT_r[...], owT_r[...])
    C2 = jnp.dot(C1, ewT_r[...])
    C3 = jnp.dot(C2, pwT_r[...])
    cvec = (jnp.dot(jnp.dot(bv_r[...], owT_r[...]) + ob_r[...], ewT_r[...])
            + eb_r[...])
    cvec = jnp.dot(cvec, pwT_r[...]) + pb_r[...]
    Ms = jnp.dot(C3, aT_r[...])
    Md = jnp.dot(C3, bT_r[...])
    bias = (jnp.dot(cvec, aT_r[...]) + jnp.dot(cvec, bT_r[...]) + lb1_r[...]
            + jnp.dot(tb2_r[...], Ms[256:384]) + jnp.dot(tb2_r[...], Md[256:384]))
    T2s = jnp.dot(tw2T_r[...], Ms[256:384])
    T2d = jnp.dot(tw2T_r[...], Md[256:384])

    ts = ts_r[...]
    found = jdp_r[...] > 0
    mem2_s = nhs_r[...]
    lu2_s = tjs_r[...]
    mem2_d = jnp.where(found, nhd_r[...], memd_r[...])
    lu2_d = jnp.where(found, tjd_r[...], lud_r[...])
    te_s = jax.nn.relu((ts - lu2_s) * tw1_r[...] + tb1_r[...])
    te_d = jax.nn.relu((ts - lu2_d) * tw1_r[...] + tb1_r[...])
    x1 = jax.nn.relu(
        jnp.dot(mem2_s, Ms[:128]) + jnp.dot(nfs_r[...], Ms[128:256])
        + jnp.dot(te_s, T2s)
        + jnp.dot(mem2_d, Md[:128]) + jnp.dot(nfd_r[...], Md[128:256])
        + jnp.dot(te_d, T2d) + bias)
    x2 = jax.nn.relu(jnp.dot(x1, w2T_r[...]) + lb2_r[...])
    out_r[...] = jnp.dot(x2, w3T_r[...]) + lb3_r[...]


def kernel(source_nodes, destination_nodes, timestamps, edge_idxs,
           node_features, edge_features, memory, last_update,
           t_w1, t_b1, t_w2, t_b2,
           msg_w1, msg_b1, msg_w2, msg_b2,
           gru_wih, gru_whh, gru_bih, gru_bhh,
           attn_in_w, attn_in_b, attn_out_w, attn_out_b,
           emb_w, emb_b, proj_w, proj_b,
           lp_w1, lp_b1, lp_w2, lp_b2, lp_w3, lp_b3):
    src = source_nodes.astype(jnp.int32)
    dst = destination_nodes.astype(jnp.int32)
    eidx = edge_idxs.astype(jnp.int32)
    ts = timestamps.astype(jnp.float32)
    lu2d = jnp.reshape(
        jnp.pad(last_update.astype(jnp.float32), (0, _TROWS * 128 - _N)),
        (_TROWS, 128))

    (nf_s, nf_d, ef_g, mem_s, mem_d, lu_s, lu_d, js, jd, jdp) = \
        _sc_gather_resolve(src, dst, eidx, node_features, edge_features,
                           memory, lu2d)

    ts2 = ts[:, None]
    tab = pl.pallas_call(
        _tc_msg_gru,
        out_shape=jax.ShapeDtypeStruct((_B, _F), jnp.float32),
    )(mem_s, nf_d, ef_g, ts2, lu_s[:, None],
      msg_w1[:, 0:128].T, msg_w1[:, 128:256].T, msg_w1[:, 256:384].T,
      msg_w1[:, 384][None, :], msg_w1[:, 385][None, :], msg_b1[None, :],
      msg_w2.T, msg_b2[None, :],
      gru_wih.T, gru_whh.T, gru_bih[None, :], gru_bhh[None, :])

    nh_s, nh_d, tjs, tjd = _sc_permute_rows(tab, ts, js, jd)

    D = 384
    out = pl.pallas_call(
        _tc_embed_lp,
        out_shape=jax.ShapeDtypeStruct((_B, 1), jnp.float32),
    )(nh_s, nh_d, tjs[:, None], tjd[:, None], jdp[:, None], mem_d,
      lu_d[:, None], nf_s, nf_d, ts2,
      attn_in_w[2 * D:].T, attn_in_b[2 * D:][None, :],
      attn_out_w.T, attn_out_b[None, :],
      emb_w.T, emb_b[None, :], proj_w.T, proj_b[None, :],
      t_w1[:, 0][None, :], t_b1[None, :], t_w2.T, t_b2[None, :],
      lp_w1[:, :128].T, lp_w1[:, 128:].T, lp_b1[None, :],
      lp_w2.T, lp_b2[None, :], lp_w3.T, lp_b3[None, :])
    return out
